# Initial kernel scaffold; baseline (speedup 1.0000x reference)
import jax
import jax.numpy as jnp
from jax.experimental import pallas as pl


def kernel(x, edge_index, params, batch_size):
    raise NotImplementedError("write your pallas kernel here")


# speedup vs baseline: 1.0864x; 1.0864x over previous
import functools

import jax
import jax.numpy as jnp
from jax.experimental import pallas as pl
from jax.experimental.pallas import tpu as pltpu

D = 64
N = 50000
BATCH = 100


_BLK = 2000
_NB = N // _BLK


def _bn_kernel(x_ref, g_ref, b_ref, o_ref, s_ref, q_ref):
    p = pl.program_id(0)
    x = x_ref[...]

    @pl.when(jnp.logical_and(p == 0, pl.program_id(1) == 0))
    def _init():
        s_ref[...] = jnp.zeros_like(s_ref)
        q_ref[...] = jnp.zeros_like(q_ref)

    @pl.when(p == 0)
    def _acc():
        s_ref[...] += jnp.sum(x, axis=0, keepdims=True)
        q_ref[...] += jnp.sum(x * x, axis=0, keepdims=True)

    @pl.when(p == 1)
    def _norm():
        mu = s_ref[...] / N
        var = q_ref[...] / N - mu * mu
        o_ref[...] = (x - mu) / jnp.sqrt(var + 1e-5) * g_ref[...] + b_ref[...]


def _bn_pallas(x, g, b):
    return pl.pallas_call(
        _bn_kernel,
        grid=(2, _NB),
        in_specs=[
            pl.BlockSpec((_BLK, D), lambda p, j: (j, 0)),
            pl.BlockSpec((1, D), lambda p, j: (0, 0)),
            pl.BlockSpec((1, D), lambda p, j: (0, 0)),
        ],
        out_specs=pl.BlockSpec((_BLK, D), lambda p, j: (j, 0)),
        scratch_shapes=[
            pltpu.VMEM((1, D), jnp.float32),
            pltpu.VMEM((1, D), jnp.float32),
        ],
        out_shape=jax.ShapeDtypeStruct(x.shape, x.dtype),
    )(x, g.reshape(1, D), b.reshape(1, D))


def _bn(x, g, b, eps):
    mu = x.mean(axis=0)
    var = x.var(axis=0)
    return (x - mu) / jnp.sqrt(var + eps) * g + b


def _graph_sum(src, dst, h, n):
    return jax.ops.segment_sum(h[src], dst, num_segments=n)


def _graph_max(src, dst, h, n):
    m = jax.ops.segment_max(h[src], dst, num_segments=n)
    return jnp.where(jnp.isfinite(m), m, 0.0)


def _gcn_layer(p, src, dst, n, feats, use_pallas_bn=False):
    agg = _graph_sum(src, dst, feats, n)
    h = agg @ p['W'] + p['b']
    h = jax.nn.relu(h)
    res = jax.nn.relu(feats @ p['Wr'].T + p['br'])
    h = h + res
    if use_pallas_bn:
        return _bn_pallas(h, p['g'], p['be'])
    return _bn(h, p['g'], p['be'], 1e-5)


def kernel(x, edge_index, params, batch_size):
    src = edge_index[0]
    dst = edge_index[1]
    n = x.shape[0]
    h = x @ params['W_init'].T
    s = h @ params['s_W1'].T + params['s_b1']
    s = jax.nn.gelu(s, approximate=False)
    s = _bn(s, params['s_g1'], params['s_be1'], 0.1)
    s = s @ params['s_W2'].T + params['s_b2']
    s = jax.nn.gelu(s, approximate=False)
    r = _bn(s, params['s_g2'], params['s_be2'], 0.1)
    h = r + s
    result = jnp.zeros_like(h)
    for i in range(2):
        h = _gcn_layer(params['gnn'][i], src, dst, n, h)
        pooled = _graph_max(src, dst, h, n)
        result = result + _gcn_layer(params['smooth'][i], src, dst, n, pooled)
    result = _gcn_layer(params['smooth_final'], src, dst, n, result, use_pallas_bn=True)
    return result.reshape(BATCH, -1, D)


# speedup vs baseline: 1.8702x; 1.7214x over previous
import functools

import jax
import jax.numpy as jnp
from jax import lax
from jax.experimental import pallas as pl
from jax.experimental.pallas import tpu as pltpu
from jax.experimental.pallas import tpu_sc as plsc

D = 64
DH = 32
N = 50000
E = 800000
BATCH = 100

_NTILES = 16
_EPT = E // _NTILES
_CHUNK = 128
_NFULL = _EPT // _CHUNK
_TAIL = _EPT - _NFULL * _CHUNK
_NPT = 3128
_NPT_LAST = N - _NPT * (_NTILES - 1)


def _segsum_body(hlo, hhi, src, dst, zeros, out_lo, out_hi,
                 idx_v, dst_v, rows_v, idx_t, dst_t, rows_t, acc, sem):
    c = lax.axis_index("c")
    s = lax.axis_index("s")
    @pl.when(s < _NTILES - 1)
    def _():
        pltpu.sync_copy(zeros.at[pl.ds(s * _NPT, _NPT)], acc.at[pl.ds(s * _NPT, _NPT)])

    @pl.when(s == _NTILES - 1)
    def _():
        pltpu.sync_copy(zeros.at[pl.ds(s * _NPT, _NPT_LAST)],
                        acc.at[pl.ds(s * _NPT, _NPT_LAST)])

    plsc.subcore_barrier()

    ebase = s * _EPT

    def chunk(off, idx_ref, dst_ref, rows_ref, count):
        pltpu.sync_copy(src.at[pl.ds(off, count)], idx_ref)
        pltpu.sync_copy(dst.at[pl.ds(off, count)], dst_ref)

        @pl.when(c == 0)
        def _():
            pltpu.async_copy(hlo.at[idx_ref], rows_ref, sem).wait()

        @pl.when(c == 1)
        def _():
            pltpu.async_copy(hhi.at[idx_ref], rows_ref, sem).wait()

        pltpu.sync_copy(rows_ref, acc.at[dst_ref], add=True)

    def body(g, carry):
        chunk(ebase + g * _CHUNK, idx_v, dst_v, rows_v, _CHUNK)
        return carry

    lax.fori_loop(0, _NFULL, body, 0)
    chunk(ebase + _NFULL * _CHUNK, idx_t, dst_t, rows_t, _TAIL)

    plsc.subcore_barrier()

    def writeout(out_ref):
        @pl.when(s < _NTILES - 1)
        def _():
            pltpu.sync_copy(acc.at[pl.ds(s * _NPT, _NPT)],
                            out_ref.at[pl.ds(s * _NPT, _NPT)])

        @pl.when(s == _NTILES - 1)
        def _():
            pltpu.sync_copy(acc.at[pl.ds(s * _NPT, _NPT_LAST)],
                            out_ref.at[pl.ds(s * _NPT, _NPT_LAST)])

    @pl.when(c == 0)
    def _():
        writeout(out_lo)

    @pl.when(c == 1)
    def _():
        writeout(out_hi)


_segsum_call = functools.partial(
    pl.kernel,
    mesh=plsc.VectorSubcoreMesh(core_axis_name="c", subcore_axis_name="s"),
    compiler_params=pltpu.CompilerParams(use_tc_tiling_on_sc=False),
    out_type=[jax.ShapeDtypeStruct((N, DH), jnp.float32),
              jax.ShapeDtypeStruct((N, DH), jnp.float32)],
    scratch_types=[
        pltpu.VMEM((_CHUNK,), jnp.int32),
        pltpu.VMEM((_CHUNK,), jnp.int32),
        pltpu.VMEM((_CHUNK, DH), jnp.float32),
        pltpu.VMEM((_TAIL,), jnp.int32),
        pltpu.VMEM((_TAIL,), jnp.int32),
        pltpu.VMEM((_TAIL, DH), jnp.float32),
        pltpu.VMEM_SHARED((N, DH), jnp.float32),
        pltpu.SemaphoreType.DMA,
    ],
)(_segsum_body)


def _graph_sum_sc(src, dst, h):
    h_lo = h[:, :DH]
    h_hi = h[:, DH:]
    zeros = jnp.zeros((N, DH), jnp.float32)
    out_lo, out_hi = _segsum_call(h_lo, h_hi, src, dst, zeros)
    return jnp.concatenate([out_lo, out_hi], axis=1)


_BLK = 2000
_NB = N // _BLK


def _bn_kernel(x_ref, g_ref, b_ref, o_ref, s_ref, q_ref):
    p = pl.program_id(0)
    x = x_ref[...]

    @pl.when(jnp.logical_and(p == 0, pl.program_id(1) == 0))
    def _init():
        s_ref[...] = jnp.zeros_like(s_ref)
        q_ref[...] = jnp.zeros_like(q_ref)

    @pl.when(p == 0)
    def _acc():
        s_ref[...] += jnp.sum(x, axis=0, keepdims=True)
        q_ref[...] += jnp.sum(x * x, axis=0, keepdims=True)

    @pl.when(p == 1)
    def _norm():
        mu = s_ref[...] / N
        var = q_ref[...] / N - mu * mu
        o_ref[...] = (x - mu) / jnp.sqrt(var + 1e-5) * g_ref[...] + b_ref[...]


def _bn_pallas(x, g, b):
    return pl.pallas_call(
        _bn_kernel,
        grid=(2, _NB),
        in_specs=[
            pl.BlockSpec((_BLK, D), lambda p, j: (j, 0)),
            pl.BlockSpec((1, D), lambda p, j: (0, 0)),
            pl.BlockSpec((1, D), lambda p, j: (0, 0)),
        ],
        out_specs=pl.BlockSpec((_BLK, D), lambda p, j: (j, 0)),
        scratch_shapes=[
            pltpu.VMEM((1, D), jnp.float32),
            pltpu.VMEM((1, D), jnp.float32),
        ],
        out_shape=jax.ShapeDtypeStruct(x.shape, x.dtype),
    )(x, g.reshape(1, D), b.reshape(1, D))


def _bn(x, g, b, eps):
    mu = x.mean(axis=0)
    var = x.var(axis=0)
    return (x - mu) / jnp.sqrt(var + eps) * g + b


def _graph_sum(src, dst, h, n):
    return _graph_sum_sc(src, dst, h)


def _graph_max(src, dst, h, n):
    m = jax.ops.segment_max(h[src], dst, num_segments=n)
    return jnp.where(jnp.isfinite(m), m, 0.0)


def _gcn_layer(p, src, dst, n, feats, use_pallas_bn=False):
    agg = _graph_sum(src, dst, feats, n)
    h = agg @ p['W'] + p['b']
    h = jax.nn.relu(h)
    res = jax.nn.relu(feats @ p['Wr'].T + p['br'])
    h = h + res
    if use_pallas_bn:
        return _bn_pallas(h, p['g'], p['be'])
    return _bn(h, p['g'], p['be'], 1e-5)


def kernel(x, edge_index, params, batch_size):
    src = edge_index[0]
    dst = edge_index[1]
    n = x.shape[0]
    h = x @ params['W_init'].T
    s = h @ params['s_W1'].T + params['s_b1']
    s = jax.nn.gelu(s, approximate=False)
    s = _bn(s, params['s_g1'], params['s_be1'], 0.1)
    s = s @ params['s_W2'].T + params['s_b2']
    s = jax.nn.gelu(s, approximate=False)
    r = _bn(s, params['s_g2'], params['s_be2'], 0.1)
    h = r + s
    result = jnp.zeros_like(h)
    for i in range(2):
        h = _gcn_layer(params['gnn'][i], src, dst, n, h)
        pooled = _graph_max(src, dst, h, n)
        result = result + _gcn_layer(params['smooth'][i], src, dst, n, pooled)
    result = _gcn_layer(params['smooth_final'], src, dst, n, result, use_pallas_bn=True)
    return result.reshape(BATCH, -1, D)


# speedup vs baseline: 2.2713x; 1.2145x over previous
import functools

import jax
import jax.numpy as jnp
from jax import lax
from jax.experimental import pallas as pl
from jax.experimental.pallas import tpu as pltpu
from jax.experimental.pallas import tpu_sc as plsc

D = 64
DH = 32
N = 50000
E = 800000
BATCH = 100

_NTILES = 16
_EPT = E // _NTILES
_CHUNK = 400
_NFULL = _EPT // _CHUNK
_NPT = 3128
_NPT_LAST = N - _NPT * (_NTILES - 1)


def _segsum_body(hlo, hhi, src, dst, zeros, out_lo, out_hi,
                 idx_v, dst_v, rows_v, acc, sem):
    c = lax.axis_index("c")
    s = lax.axis_index("s")
    @pl.when(s < _NTILES - 1)
    def _():
        pltpu.sync_copy(zeros.at[pl.ds(s * _NPT, _NPT)], acc.at[pl.ds(s * _NPT, _NPT)])

    @pl.when(s == _NTILES - 1)
    def _():
        pltpu.sync_copy(zeros.at[pl.ds(s * _NPT, _NPT_LAST)],
                        acc.at[pl.ds(s * _NPT, _NPT_LAST)])

    plsc.subcore_barrier()

    ebase = s * _EPT

    def chunk(off, idx_ref, dst_ref, rows_ref, count):
        pltpu.sync_copy(src.at[pl.ds(off, count)], idx_ref)
        pltpu.sync_copy(dst.at[pl.ds(off, count)], dst_ref)

        @pl.when(c == 0)
        def _():
            pltpu.async_copy(hlo.at[idx_ref], rows_ref, sem).wait()

        @pl.when(c == 1)
        def _():
            pltpu.async_copy(hhi.at[idx_ref], rows_ref, sem).wait()

        pltpu.sync_copy(rows_ref, acc.at[dst_ref], add=True)

    def body(g, carry):
        chunk(ebase + g * _CHUNK, idx_v, dst_v, rows_v, _CHUNK)
        return carry

    lax.fori_loop(0, _NFULL, body, 0)

    plsc.subcore_barrier()

    def writeout(out_ref):
        @pl.when(s < _NTILES - 1)
        def _():
            pltpu.sync_copy(acc.at[pl.ds(s * _NPT, _NPT)],
                            out_ref.at[pl.ds(s * _NPT, _NPT)])

        @pl.when(s == _NTILES - 1)
        def _():
            pltpu.sync_copy(acc.at[pl.ds(s * _NPT, _NPT_LAST)],
                            out_ref.at[pl.ds(s * _NPT, _NPT_LAST)])

    @pl.when(c == 0)
    def _():
        writeout(out_lo)

    @pl.when(c == 1)
    def _():
        writeout(out_hi)


_segsum_call = functools.partial(
    pl.kernel,
    mesh=plsc.VectorSubcoreMesh(core_axis_name="c", subcore_axis_name="s"),
    compiler_params=pltpu.CompilerParams(use_tc_tiling_on_sc=False),
    out_type=[jax.ShapeDtypeStruct((N, DH), jnp.float32),
              jax.ShapeDtypeStruct((N, DH), jnp.float32)],
    scratch_types=[
        pltpu.VMEM((_CHUNK,), jnp.int32),
        pltpu.VMEM((_CHUNK,), jnp.int32),
        pltpu.VMEM((_CHUNK, DH), jnp.float32),
        pltpu.VMEM_SHARED((N, DH), jnp.float32),
        pltpu.SemaphoreType.DMA,
    ],
)(_segsum_body)


def _graph_sum_sc(src, dst, h):
    h_lo = h[:, :DH]
    h_hi = h[:, DH:]
    zeros = jnp.zeros((N, DH), jnp.float32)
    out_lo, out_hi = _segsum_call(h_lo, h_hi, src, dst, zeros)
    return jnp.concatenate([out_lo, out_hi], axis=1)


_BLK = 2000
_NB = N // _BLK


def _bn_kernel(x_ref, g_ref, b_ref, o_ref, s_ref, q_ref):
    p = pl.program_id(0)
    x = x_ref[...]

    @pl.when(jnp.logical_and(p == 0, pl.program_id(1) == 0))
    def _init():
        s_ref[...] = jnp.zeros_like(s_ref)
        q_ref[...] = jnp.zeros_like(q_ref)

    @pl.when(p == 0)
    def _acc():
        s_ref[...] += jnp.sum(x, axis=0, keepdims=True)
        q_ref[...] += jnp.sum(x * x, axis=0, keepdims=True)

    @pl.when(p == 1)
    def _norm():
        mu = s_ref[...] / N
        var = q_ref[...] / N - mu * mu
        o_ref[...] = (x - mu) / jnp.sqrt(var + 1e-5) * g_ref[...] + b_ref[...]


def _bn_pallas(x, g, b):
    return pl.pallas_call(
        _bn_kernel,
        grid=(2, _NB),
        in_specs=[
            pl.BlockSpec((_BLK, D), lambda p, j: (j, 0)),
            pl.BlockSpec((1, D), lambda p, j: (0, 0)),
            pl.BlockSpec((1, D), lambda p, j: (0, 0)),
        ],
        out_specs=pl.BlockSpec((_BLK, D), lambda p, j: (j, 0)),
        scratch_shapes=[
            pltpu.VMEM((1, D), jnp.float32),
            pltpu.VMEM((1, D), jnp.float32),
        ],
        out_shape=jax.ShapeDtypeStruct(x.shape, x.dtype),
    )(x, g.reshape(1, D), b.reshape(1, D))


def _bn(x, g, b, eps):
    mu = x.mean(axis=0)
    var = x.var(axis=0)
    return (x - mu) / jnp.sqrt(var + eps) * g + b


def _graph_sum(src, dst, h, n):
    return _graph_sum_sc(src, dst, h)


def _graph_max(src, dst, h, n):
    m = jax.ops.segment_max(h[src], dst, num_segments=n)
    return jnp.where(jnp.isfinite(m), m, 0.0)


def _gcn_layer(p, src, dst, n, feats, use_pallas_bn=False):
    agg = _graph_sum(src, dst, feats, n)
    h = agg @ p['W'] + p['b']
    h = jax.nn.relu(h)
    res = jax.nn.relu(feats @ p['Wr'].T + p['br'])
    h = h + res
    if use_pallas_bn:
        return _bn_pallas(h, p['g'], p['be'])
    return _bn(h, p['g'], p['be'], 1e-5)


def kernel(x, edge_index, params, batch_size):
    src = edge_index[0]
    dst = edge_index[1]
    n = x.shape[0]
    h = x @ params['W_init'].T
    s = h @ params['s_W1'].T + params['s_b1']
    s = jax.nn.gelu(s, approximate=False)
    s = _bn(s, params['s_g1'], params['s_be1'], 0.1)
    s = s @ params['s_W2'].T + params['s_b2']
    s = jax.nn.gelu(s, approximate=False)
    r = _bn(s, params['s_g2'], params['s_be2'], 0.1)
    h = r + s
    result = jnp.zeros_like(h)
    for i in range(2):
        h = _gcn_layer(params['gnn'][i], src, dst, n, h)
        pooled = _graph_max(src, dst, h, n)
        result = result + _gcn_layer(params['smooth'][i], src, dst, n, pooled)
    result = _gcn_layer(params['smooth_final'], src, dst, n, result, use_pallas_bn=True)
    return result.reshape(BATCH, -1, D)
